# Initial kernel scaffold; baseline (speedup 1.0000x reference)
import jax
import jax.numpy as jnp
from jax.experimental import pallas as pl


def kernel(x, indices_s, weight1, weight2, bias1, bias2):
    raise NotImplementedError("write your pallas kernel here")


# speedup vs baseline: 4.3384x; 4.3384x over previous
import functools

import jax
import jax.numpy as jnp
from jax.experimental import pallas as pl
from jax.experimental.pallas import tpu as pltpu

ROWS = 8


def _ffn_kernel(be_ref, tok_ref, valid_ref, x_ref, w1_ref, w2_ref, b1_ref,
                b2_ref, out_ref):
    b = pl.program_id(0)
    e = be_ref[b]
    base = b * ROWS
    rows = [x_ref[pl.ds(tok_ref[base + i], 1), :] for i in range(ROWS)]
    xb = jnp.concatenate(rows, axis=0)
    h = jnp.dot(xb, w1_ref[0], preferred_element_type=jnp.float32)
    h = h + b1_ref[pl.ds(e, 1), :]
    h = 0.5 * h * (1.0 + jax.lax.erf(h * 0.7071067811865476))
    o = jnp.dot(h, w2_ref[0], preferred_element_type=jnp.float32)
    o = o + b2_ref[pl.ds(e, 1), :]
    for i in range(ROWS):
        @pl.when(valid_ref[base + i] == 1)
        def _():
            out_ref[pl.ds(tok_ref[base + i], 1), :] = o[i:i + 1, :]


def kernel(x, indices_s, weight1, weight2, bias1, bias2):
    T, D = x.shape
    E, _, H = weight1.shape
    NB = T // ROWS + E

    idx = indices_s.astype(jnp.int32)
    sort_tok = jnp.argsort(idx, stable=True).astype(jnp.int32)
    sorted_e = idx[sort_tok]
    counts = jnp.bincount(idx, length=E)
    nb = (counts + ROWS - 1) // ROWS
    bend = jnp.cumsum(nb)
    bstart = bend - nb
    cstart = jnp.cumsum(counts) - counts
    rank = jnp.arange(T, dtype=jnp.int32) - cstart[sorted_e].astype(jnp.int32)
    pos = bstart[sorted_e].astype(jnp.int32) * ROWS + rank
    tok_at = jnp.zeros((NB * ROWS,), jnp.int32).at[pos].set(sort_tok)
    valid = jnp.zeros((NB * ROWS,), jnp.int32).at[pos].set(1)
    used = bend[-1].astype(jnp.int32)
    eb = jnp.searchsorted(bend, jnp.arange(NB), side='right')
    eb = jnp.minimum(eb, E - 1).astype(jnp.int32)
    e_last = jnp.take(eb, used - 1)
    block_e = jnp.where(jnp.arange(NB) < used, eb, e_last)

    grid_spec = pltpu.PrefetchScalarGridSpec(
        num_scalar_prefetch=3,
        grid=(NB,),
        in_specs=[
            pl.BlockSpec((T, D), lambda b, be, tok, va: (0, 0)),
            pl.BlockSpec((1, D, H), lambda b, be, tok, va: (be[b], 0, 0)),
            pl.BlockSpec((1, H, D), lambda b, be, tok, va: (be[b], 0, 0)),
            pl.BlockSpec((E, H), lambda b, be, tok, va: (0, 0)),
            pl.BlockSpec((E, D), lambda b, be, tok, va: (0, 0)),
        ],
        out_specs=pl.BlockSpec((T, D), lambda b, be, tok, va: (0, 0)),
    )
    out = pl.pallas_call(
        _ffn_kernel,
        grid_spec=grid_spec,
        out_shape=jax.ShapeDtypeStruct((T, D), jnp.float32),
    )(block_e, tok_at, valid, x, weight1, weight2, bias1, bias2)
    return out[:, None, :]
